# baseline (speedup 1.0000x reference)
import functools

import jax
import jax.numpy as jnp
from jax import lax
from jax.experimental import pallas as pl
from jax.experimental.pallas import tpu as pltpu
from jax.experimental.pallas import tpu_sc as plsc

B, N, D = 8, 16384, 64
P, L = 1024, 8
H = 64
BP = B * P
ITEMS = BP * L

NC, NS = 2, 16
NW = NC * NS
PER_W = ITEMS // NW
CH = 128
NCHUNK = PER_W // CH


@functools.cache
def _make_sc_gather():
    mesh = plsc.VectorSubcoreMesh(core_axis_name="c", subcore_axis_name="s")

    @functools.partial(
        pl.kernel,
        mesh=mesh,
        out_type=(
            jax.ShapeDtypeStruct((ITEMS, D), jnp.float32),
            jax.ShapeDtypeStruct((ITEMS, D), jnp.float32),
        ),
        scratch_types=[
            pltpu.VMEM((NCHUNK, CH), jnp.int32),
            pltpu.VMEM((NCHUNK, CH), jnp.int32),
            pltpu.VMEM((CH, D), jnp.float32),
            pltpu.VMEM((CH, D), jnp.float32),
            pltpu.SemaphoreType.DMA,
            pltpu.SemaphoreType.DMA,
        ],
        compiler_params=pltpu.CompilerParams(use_tc_tiling_on_sc=False),
    )
    def sc_gather(graph_hbm, rel_hbm, idxn_hbm, idxr_hbm, outn_hbm, outr_hbm,
                  idxn_v, idxr_v, bufn, bufr, semn, semr):
        wid = lax.axis_index("s") * NC + lax.axis_index("c")
        row0 = wid * NCHUNK
        pltpu.sync_copy(idxn_hbm.at[pl.ds(row0, NCHUNK)], idxn_v)
        pltpu.sync_copy(idxr_hbm.at[pl.ds(row0, NCHUNK)], idxr_v)
        base = wid * PER_W

        def body(i, carry):
            cn = pltpu.async_copy(graph_hbm.at[idxn_v.at[i]], bufn, semn)
            cr = pltpu.async_copy(rel_hbm.at[idxr_v.at[i]], bufr, semr)
            cn.wait()
            cr.wait()
            pltpu.sync_copy(bufn, outn_hbm.at[pl.ds(base + i * CH, CH)])
            pltpu.sync_copy(bufr, outr_hbm.at[pl.ds(base + i * CH, CH)])
            return carry

        lax.fori_loop(0, NCHUNK, body, 0)

    return sc_gather


BLK = 1024
_PREC = lax.Precision.DEFAULT


def _lstm_body(node_ref, rel_ref, wT_ref, whhT_ref, bias_ref, out_ref):
    x = jnp.concatenate([node_ref[...], rel_ref[...]], axis=-1)
    x = x.reshape(BLK * L, 2 * D).astype(jnp.bfloat16)
    gx = jnp.dot(x, wT_ref[...].astype(jnp.bfloat16),
                 preferred_element_type=jnp.float32)
    gx = (gx + bias_ref[...]).reshape(BLK, L, 4 * H)
    whhT = whhT_ref[...].astype(jnp.bfloat16)
    h = jnp.zeros((BLK, H), jnp.float32)
    c = jnp.zeros((BLK, H), jnp.float32)
    for l in range(L):
        g = gx[:, l, :] + jnp.dot(h.astype(jnp.bfloat16), whhT,
                                  preferred_element_type=jnp.float32)
        i = jax.nn.sigmoid(g[:, 0:H])
        f = jax.nn.sigmoid(g[:, H:2 * H])
        gg = jnp.tanh(g[:, 2 * H:3 * H])
        o = jax.nn.sigmoid(g[:, 3 * H:4 * H])
        c = f * c + i * gg
        h = o * jnp.tanh(c)
    out_ref[...] = h


def _attn_body(po_ref, nopt_ref, wqT_ref, bq_ref, wkT_ref, wvT_ref, bk_ref,
               bv_ref, woT_ref, bo_ref, out_ref):
    po = po_ref[...]
    k = jnp.dot(po, wkT_ref[...], precision=_PREC) + bk_ref[...]
    v = jnp.dot(po, wvT_ref[...], precision=_PREC) + bv_ref[...]
    q = jnp.dot(nopt_ref[...], wqT_ref[...], precision=_PREC) + bq_ref[...]
    k3 = k.reshape(B, P, H)
    scores = jnp.sum(q[:, None, :] * k3, axis=-1) * 0.125
    m = jnp.max(scores, axis=-1, keepdims=True)
    e = jnp.exp(scores - m)
    w = e / jnp.sum(e, axis=-1, keepdims=True)
    attn = jnp.sum(w[:, :, None] * v.reshape(B, P, H), axis=1)
    out_ref[...] = jnp.dot(attn, woT_ref[...], precision=_PREC) + bo_ref[...]


def kernel(graph_embed, paths_nodes, paths_rels, node_opt, rel_embeds,
           W_ih, W_hh, b_ih, b_hh, in_proj_w, in_proj_b, out_proj_w, out_proj_b):
    graph_flat = graph_embed.reshape(B * N, D)
    idx_n = (paths_nodes.astype(jnp.int32)
             + (jnp.arange(B, dtype=jnp.int32) * N)[:, None, None])
    idx_n = idx_n.reshape(ITEMS // CH, CH)
    idx_r = paths_rels.astype(jnp.int32).reshape(ITEMS // CH, CH)

    node_g, rel_g = _make_sc_gather()(graph_flat, rel_embeds, idx_n, idx_r)
    node_seq = node_g.reshape(BP, L, D)
    rel_seq = rel_g.reshape(BP, L, D)

    wT = W_ih.T
    whhT = W_hh.T
    bias = (b_ih + b_hh)[None, :]

    paths_out = pl.pallas_call(
        _lstm_body,
        grid=(BP // BLK,),
        in_specs=[
            pl.BlockSpec((BLK, L, D), lambda i: (i, 0, 0)),
            pl.BlockSpec((BLK, L, D), lambda i: (i, 0, 0)),
            pl.BlockSpec((2 * D, 4 * H), lambda i: (0, 0)),
            pl.BlockSpec((H, 4 * H), lambda i: (0, 0)),
            pl.BlockSpec((1, 4 * H), lambda i: (0, 0)),
        ],
        out_specs=pl.BlockSpec((BLK, H), lambda i: (i, 0)),
        out_shape=jax.ShapeDtypeStruct((BP, H), jnp.float32),
    )(node_seq, rel_seq, wT, whhT, bias)

    Wq, Wk, Wv = in_proj_w[:H], in_proj_w[H:2 * H], in_proj_w[2 * H:]
    bq, bk, bv = in_proj_b[:H], in_proj_b[H:2 * H], in_proj_b[2 * H:]

    out = pl.pallas_call(
        _attn_body,
        in_specs=[
            pl.BlockSpec((BP, H), lambda: (0, 0)),
            pl.BlockSpec((B, H), lambda: (0, 0)),
            pl.BlockSpec((H, H), lambda: (0, 0)),
            pl.BlockSpec((1, H), lambda: (0, 0)),
            pl.BlockSpec((H, H), lambda: (0, 0)),
            pl.BlockSpec((H, H), lambda: (0, 0)),
            pl.BlockSpec((1, H), lambda: (0, 0)),
            pl.BlockSpec((1, H), lambda: (0, 0)),
            pl.BlockSpec((H, H), lambda: (0, 0)),
            pl.BlockSpec((1, H), lambda: (0, 0)),
        ],
        out_specs=pl.BlockSpec((B, H), lambda: (0, 0)),
        out_shape=jax.ShapeDtypeStruct((B, H), jnp.float32),
    )(paths_out, node_opt, Wq.T, bq[None, :], Wk.T, Wv.T, bk[None, :],
      bv[None, :], out_proj_w.T, out_proj_b[None, :])
    return out


# speedup vs baseline: 1.7944x; 1.7944x over previous
import functools

import jax
import jax.numpy as jnp
from jax import lax
from jax.experimental import pallas as pl
from jax.experimental.pallas import tpu as pltpu
from jax.experimental.pallas import tpu_sc as plsc

B, N, D = 8, 16384, 64
P, L = 1024, 8
H = 64
BP = B * P
ITEMS = BP * L

NC, NS = 2, 16
NW = NC * NS
PER_W = ITEMS // NW
CH = 128
NCHUNK = PER_W // CH


@functools.cache
def _make_sc_gather():
    mesh = plsc.VectorSubcoreMesh(core_axis_name="c", subcore_axis_name="s")

    @functools.partial(
        pl.kernel,
        mesh=mesh,
        out_type=(
            jax.ShapeDtypeStruct((ITEMS, D), jnp.float32),
            jax.ShapeDtypeStruct((ITEMS, D), jnp.float32),
        ),
        scratch_types=[
            pltpu.VMEM((NCHUNK, CH), jnp.int32),
            pltpu.VMEM((NCHUNK, CH), jnp.int32),
            pltpu.VMEM((CH, D), jnp.float32),
            pltpu.VMEM((CH, D), jnp.float32),
            pltpu.SemaphoreType.DMA,
            pltpu.SemaphoreType.DMA,
        ],
        compiler_params=pltpu.CompilerParams(use_tc_tiling_on_sc=False),
    )
    def sc_gather(graph_hbm, rel_hbm, idxn_hbm, idxr_hbm, outn_hbm, outr_hbm,
                  idxn_v, idxr_v, bufn, bufr, semn, semr):
        wid = lax.axis_index("s") * NC + lax.axis_index("c")
        row0 = wid * NCHUNK
        pltpu.sync_copy(idxn_hbm.at[pl.ds(row0, NCHUNK)], idxn_v)
        pltpu.sync_copy(idxr_hbm.at[pl.ds(row0, NCHUNK)], idxr_v)
        base = wid * PER_W

        def body(i, carry):
            cn = pltpu.async_copy(graph_hbm.at[idxn_v.at[i]], bufn, semn)
            cr = pltpu.async_copy(rel_hbm.at[idxr_v.at[i]], bufr, semr)
            cn.wait()
            cr.wait()
            pltpu.sync_copy(bufn, outn_hbm.at[pl.ds(base + i * CH, CH)])
            pltpu.sync_copy(bufr, outr_hbm.at[pl.ds(base + i * CH, CH)])
            return carry

        lax.fori_loop(0, NCHUNK, body, 0)

    return sc_gather


BLK = 1024
_PREC = lax.Precision.DEFAULT


def _lstm_body(node_ref, rel_ref, wnT_ref, wrT_ref, whhT_ref, bias_ref, out_ref):
    xn = node_ref[...].reshape(L * BLK, D).astype(jnp.bfloat16)
    xr = rel_ref[...].reshape(L * BLK, D).astype(jnp.bfloat16)
    gx = (jnp.dot(xn, wnT_ref[...], preferred_element_type=jnp.float32)
          + jnp.dot(xr, wrT_ref[...], preferred_element_type=jnp.float32)
          + bias_ref[...])
    whhT = whhT_ref[...]
    h = jnp.zeros((BLK, H), jnp.float32)
    c = jnp.zeros((BLK, H), jnp.float32)
    for l in range(L):
        g = gx[l * BLK:(l + 1) * BLK] + jnp.dot(
            h.astype(jnp.bfloat16), whhT, preferred_element_type=jnp.float32)
        i = jax.nn.sigmoid(g[:, 0:H])
        f = jax.nn.sigmoid(g[:, H:2 * H])
        gg = jnp.tanh(g[:, 2 * H:3 * H])
        o = jax.nn.sigmoid(g[:, 3 * H:4 * H])
        c = f * c + i * gg
        h = o * jnp.tanh(c)
    out_ref[...] = h


def _attn_body(po_ref, nopt_ref, wqT_ref, bq_ref, wkT_ref, wvT_ref, bk_ref,
               bv_ref, woT_ref, bo_ref, out_ref):
    po = po_ref[...]
    k = jnp.dot(po, wkT_ref[...], precision=_PREC) + bk_ref[...]
    v = jnp.dot(po, wvT_ref[...], precision=_PREC) + bv_ref[...]
    q = jnp.dot(nopt_ref[...], wqT_ref[...], precision=_PREC) + bq_ref[...]
    k3 = k.reshape(B, P, H)
    scores = jnp.sum(q[:, None, :] * k3, axis=-1) * 0.125
    m = jnp.max(scores, axis=-1, keepdims=True)
    e = jnp.exp(scores - m)
    w = e / jnp.sum(e, axis=-1, keepdims=True)
    attn = jnp.sum(w[:, :, None] * v.reshape(B, P, H), axis=1)
    out_ref[...] = jnp.dot(attn, woT_ref[...], precision=_PREC) + bo_ref[...]


def kernel(graph_embed, paths_nodes, paths_rels, node_opt, rel_embeds,
           W_ih, W_hh, b_ih, b_hh, in_proj_w, in_proj_b, out_proj_w, out_proj_b):
    graph_flat = graph_embed.reshape(B * N, D)
    idx_n = (jnp.transpose(paths_nodes, (2, 0, 1)).astype(jnp.int32)
             + (jnp.arange(B, dtype=jnp.int32) * N)[None, :, None])
    idx_n = idx_n.reshape(ITEMS // CH, CH)
    idx_r = jnp.transpose(paths_rels, (2, 0, 1)).astype(jnp.int32)
    idx_r = idx_r.reshape(ITEMS // CH, CH)

    node_g, rel_g = _make_sc_gather()(graph_flat, rel_embeds, idx_n, idx_r)
    node_seq = node_g.reshape(L, BP, D)
    rel_seq = rel_g.reshape(L, BP, D)

    wnT = W_ih[:, :D].T.astype(jnp.bfloat16)
    wrT = W_ih[:, D:].T.astype(jnp.bfloat16)
    whhT = W_hh.T.astype(jnp.bfloat16)
    bias = (b_ih + b_hh)[None, :]

    paths_out = pl.pallas_call(
        _lstm_body,
        grid=(BP // BLK,),
        in_specs=[
            pl.BlockSpec((L, BLK, D), lambda i: (0, i, 0)),
            pl.BlockSpec((L, BLK, D), lambda i: (0, i, 0)),
            pl.BlockSpec((D, 4 * H), lambda i: (0, 0)),
            pl.BlockSpec((D, 4 * H), lambda i: (0, 0)),
            pl.BlockSpec((H, 4 * H), lambda i: (0, 0)),
            pl.BlockSpec((1, 4 * H), lambda i: (0, 0)),
        ],
        out_specs=pl.BlockSpec((BLK, H), lambda i: (i, 0)),
        out_shape=jax.ShapeDtypeStruct((BP, H), jnp.float32),
    )(node_seq, rel_seq, wnT, wrT, whhT, bias)

    Wq, Wk, Wv = in_proj_w[:H], in_proj_w[H:2 * H], in_proj_w[2 * H:]
    bq, bk, bv = in_proj_b[:H], in_proj_b[H:2 * H], in_proj_b[2 * H:]

    out = pl.pallas_call(
        _attn_body,
        in_specs=[
            pl.BlockSpec((BP, H), lambda: (0, 0)),
            pl.BlockSpec((B, H), lambda: (0, 0)),
            pl.BlockSpec((H, H), lambda: (0, 0)),
            pl.BlockSpec((1, H), lambda: (0, 0)),
            pl.BlockSpec((H, H), lambda: (0, 0)),
            pl.BlockSpec((H, H), lambda: (0, 0)),
            pl.BlockSpec((1, H), lambda: (0, 0)),
            pl.BlockSpec((1, H), lambda: (0, 0)),
            pl.BlockSpec((H, H), lambda: (0, 0)),
            pl.BlockSpec((1, H), lambda: (0, 0)),
        ],
        out_specs=pl.BlockSpec((B, H), lambda: (0, 0)),
        out_shape=jax.ShapeDtypeStruct((B, H), jnp.float32),
    )(paths_out, node_opt, Wq.T, bq[None, :], Wk.T, Wv.T, bk[None, :],
      bv[None, :], out_proj_w.T, out_proj_b[None, :])
    return out


# speedup vs baseline: 2.3402x; 1.3041x over previous
import functools

import jax
import jax.numpy as jnp
from jax import lax
from jax.experimental import pallas as pl
from jax.experimental.pallas import tpu as pltpu
from jax.experimental.pallas import tpu_sc as plsc

B, N, D = 8, 16384, 64
P, L = 1024, 8
H = 64
BP = B * P
ITEMS = BP * L

NC, NS = 2, 16
NW = NC * NS
PER_W = ITEMS // NW
CH = 128
NCHUNK = PER_W // CH


NBUF = 4


@functools.cache
def _make_sc_gather():
    mesh = plsc.VectorSubcoreMesh(core_axis_name="c", subcore_axis_name="s")

    @functools.partial(
        pl.kernel,
        mesh=mesh,
        out_type=jax.ShapeDtypeStruct((ITEMS, 2 * D), jnp.float32),
        scratch_types=[
            pltpu.VMEM((NCHUNK, CH), jnp.int32),
            pltpu.VMEM((NCHUNK, CH), jnp.int32),
            pltpu.VMEM((NBUF, CH, D), jnp.float32),
            pltpu.VMEM((NBUF, CH, D), jnp.float32),
            pltpu.SemaphoreType.DMA,
            pltpu.SemaphoreType.DMA,
        ],
        compiler_params=pltpu.CompilerParams(use_tc_tiling_on_sc=False),
    )
    def sc_gather(graph_hbm, rel_hbm, idxn_hbm, idxr_hbm, out_hbm,
                  idxn_v, idxr_v, bufn, bufr, semn, semr):
        wid = lax.axis_index("s") * NC + lax.axis_index("c")
        row0 = wid * NCHUNK
        pltpu.sync_copy(idxn_hbm.at[pl.ds(row0, NCHUNK)], idxn_v)
        pltpu.sync_copy(idxr_hbm.at[pl.ds(row0, NCHUNK)], idxr_v)
        base = wid * PER_W

        def body(i, carry):
            copies = []
            for b in range(NBUF):
                ch = i * NBUF + b
                cn = pltpu.async_copy(graph_hbm.at[idxn_v.at[ch]], bufn.at[b], semn)
                cr = pltpu.async_copy(rel_hbm.at[idxr_v.at[ch]], bufr.at[b], semr)
                copies.append((cn, cr))
            for b in range(NBUF):
                ch = i * NBUF + b
                rows = pl.ds(base + ch * CH, CH)
                cn, cr = copies[b]
                cn.wait()
                pltpu.sync_copy(bufn.at[b], out_hbm.at[rows, pl.ds(0, D)])
                cr.wait()
                pltpu.sync_copy(bufr.at[b], out_hbm.at[rows, pl.ds(D, D)])
            return carry

        lax.fori_loop(0, NCHUNK // NBUF, body, 0)

    return sc_gather


BLK = 1024
_PREC = lax.Precision.DEFAULT


def _lstm_body(x_ref, wT_ref, whhT_ref, bias_ref, out_ref):
    x = x_ref[...].reshape(L * BLK, 2 * D).astype(jnp.bfloat16)
    gx = (jnp.dot(x, wT_ref[...], preferred_element_type=jnp.float32)
          + bias_ref[...])
    whhT = whhT_ref[...]
    h = jnp.zeros((BLK, H), jnp.float32)
    c = jnp.zeros((BLK, H), jnp.float32)
    for l in range(L):
        g = gx[l * BLK:(l + 1) * BLK] + jnp.dot(
            h.astype(jnp.bfloat16), whhT, preferred_element_type=jnp.float32)
        i = jax.nn.sigmoid(g[:, 0:H])
        f = jax.nn.sigmoid(g[:, H:2 * H])
        gg = jnp.tanh(g[:, 2 * H:3 * H])
        o = jax.nn.sigmoid(g[:, 3 * H:4 * H])
        c = f * c + i * gg
        h = o * jnp.tanh(c)
    out_ref[...] = h


def _attn_body(po_ref, nopt_ref, wqT_ref, bq_ref, wkT_ref, wvT_ref, bk_ref,
               bv_ref, woT_ref, bo_ref, out_ref):
    po = po_ref[...]
    k = jnp.dot(po, wkT_ref[...], precision=_PREC) + bk_ref[...]
    v = jnp.dot(po, wvT_ref[...], precision=_PREC) + bv_ref[...]
    q = jnp.dot(nopt_ref[...], wqT_ref[...], precision=_PREC) + bq_ref[...]
    k3 = k.reshape(B, P, H)
    scores = jnp.sum(q[:, None, :] * k3, axis=-1) * 0.125
    m = jnp.max(scores, axis=-1, keepdims=True)
    e = jnp.exp(scores - m)
    w = e / jnp.sum(e, axis=-1, keepdims=True)
    attn = jnp.sum(w[:, :, None] * v.reshape(B, P, H), axis=1)
    out_ref[...] = jnp.dot(attn, woT_ref[...], precision=_PREC) + bo_ref[...]


def kernel(graph_embed, paths_nodes, paths_rels, node_opt, rel_embeds,
           W_ih, W_hh, b_ih, b_hh, in_proj_w, in_proj_b, out_proj_w, out_proj_b):
    graph_flat = graph_embed.reshape(B * N, D)
    idx_n = (jnp.transpose(paths_nodes, (2, 0, 1)).astype(jnp.int32)
             + (jnp.arange(B, dtype=jnp.int32) * N)[None, :, None])
    idx_n = idx_n.reshape(ITEMS // CH, CH)
    idx_r = jnp.transpose(paths_rels, (2, 0, 1)).astype(jnp.int32)
    idx_r = idx_r.reshape(ITEMS // CH, CH)

    x_seq = _make_sc_gather()(graph_flat, rel_embeds, idx_n, idx_r)
    x_seq = x_seq.reshape(L, BP, 2 * D)

    wT = W_ih.T.astype(jnp.bfloat16)
    whhT = W_hh.T.astype(jnp.bfloat16)
    bias = (b_ih + b_hh)[None, :]

    paths_out = pl.pallas_call(
        _lstm_body,
        grid=(BP // BLK,),
        in_specs=[
            pl.BlockSpec((L, BLK, 2 * D), lambda i: (0, i, 0)),
            pl.BlockSpec((2 * D, 4 * H), lambda i: (0, 0)),
            pl.BlockSpec((H, 4 * H), lambda i: (0, 0)),
            pl.BlockSpec((1, 4 * H), lambda i: (0, 0)),
        ],
        out_specs=pl.BlockSpec((BLK, H), lambda i: (i, 0)),
        out_shape=jax.ShapeDtypeStruct((BP, H), jnp.float32),
    )(x_seq, wT, whhT, bias)

    Wq, Wk, Wv = in_proj_w[:H], in_proj_w[H:2 * H], in_proj_w[2 * H:]
    bq, bk, bv = in_proj_b[:H], in_proj_b[H:2 * H], in_proj_b[2 * H:]

    out = pl.pallas_call(
        _attn_body,
        in_specs=[
            pl.BlockSpec((BP, H), lambda: (0, 0)),
            pl.BlockSpec((B, H), lambda: (0, 0)),
            pl.BlockSpec((H, H), lambda: (0, 0)),
            pl.BlockSpec((1, H), lambda: (0, 0)),
            pl.BlockSpec((H, H), lambda: (0, 0)),
            pl.BlockSpec((H, H), lambda: (0, 0)),
            pl.BlockSpec((1, H), lambda: (0, 0)),
            pl.BlockSpec((1, H), lambda: (0, 0)),
            pl.BlockSpec((H, H), lambda: (0, 0)),
            pl.BlockSpec((1, H), lambda: (0, 0)),
        ],
        out_specs=pl.BlockSpec((B, H), lambda: (0, 0)),
        out_shape=jax.ShapeDtypeStruct((B, H), jnp.float32),
    )(paths_out, node_opt, Wq.T, bq[None, :], Wk.T, Wv.T, bk[None, :],
      bv[None, :], out_proj_w.T, out_proj_b[None, :])
    return out


# speedup vs baseline: 2.3626x; 1.0096x over previous
import functools

import jax
import jax.numpy as jnp
from jax import lax
from jax.experimental import pallas as pl
from jax.experimental.pallas import tpu as pltpu
from jax.experimental.pallas import tpu_sc as plsc

B, N, D = 8, 16384, 64
P, L = 1024, 8
H = 64
BP = B * P
ITEMS = BP * L

NC, NS = 2, 16
NW = NC * NS
PER_W = ITEMS // NW
CH = 128
NCHUNK = PER_W // CH


NBUF = 4


@functools.cache
def _make_sc_gather():
    mesh = plsc.VectorSubcoreMesh(core_axis_name="c", subcore_axis_name="s")

    @functools.partial(
        pl.kernel,
        mesh=mesh,
        out_type=jax.ShapeDtypeStruct((ITEMS, 2 * D), jnp.float32),
        scratch_types=[
            pltpu.VMEM((NCHUNK, CH), jnp.int32),
            pltpu.VMEM((NCHUNK, CH), jnp.int32),
            pltpu.VMEM((NBUF, CH, D), jnp.float32),
            pltpu.VMEM((NBUF, CH, D), jnp.float32),
            pltpu.SemaphoreType.DMA,
            pltpu.SemaphoreType.DMA,
        ],
        compiler_params=pltpu.CompilerParams(use_tc_tiling_on_sc=False),
    )
    def sc_gather(graph_hbm, rel_hbm, idxn_hbm, idxr_hbm, out_hbm,
                  idxn_v, idxr_v, bufn, bufr, semn, semr):
        wid = lax.axis_index("s") * NC + lax.axis_index("c")
        row0 = wid * NCHUNK
        pltpu.sync_copy(idxn_hbm.at[pl.ds(row0, NCHUNK)], idxn_v)
        pltpu.sync_copy(idxr_hbm.at[pl.ds(row0, NCHUNK)], idxr_v)
        base = wid * PER_W

        def body(i, carry):
            copies = []
            for b in range(NBUF):
                ch = i * NBUF + b
                cn = pltpu.async_copy(graph_hbm.at[idxn_v.at[ch]], bufn.at[b], semn)
                cr = pltpu.async_copy(rel_hbm.at[idxr_v.at[ch]], bufr.at[b], semr)
                copies.append((cn, cr))
            for b in range(NBUF):
                ch = i * NBUF + b
                rows = pl.ds(base + ch * CH, CH)
                cn, cr = copies[b]
                cn.wait()
                pltpu.sync_copy(bufn.at[b], out_hbm.at[rows, pl.ds(0, D)])
                cr.wait()
                pltpu.sync_copy(bufr.at[b], out_hbm.at[rows, pl.ds(D, D)])
            return carry

        lax.fori_loop(0, NCHUNK // NBUF, body, 0)

    return sc_gather


BLK = 2048
_PREC = lax.Precision.DEFAULT


def _lstm_body(x_ref, wT_ref, whhT_ref, bias_ref, out_ref):
    x = x_ref[...].reshape(L * BLK, 2 * D).astype(jnp.bfloat16)
    gx = (jnp.dot(x, wT_ref[...], preferred_element_type=jnp.float32)
          + bias_ref[...])
    whhT = whhT_ref[...]
    h = jnp.zeros((BLK, H), jnp.float32)
    c = jnp.zeros((BLK, H), jnp.float32)
    for l in range(L):
        g = gx[l * BLK:(l + 1) * BLK] + jnp.dot(
            h.astype(jnp.bfloat16), whhT, preferred_element_type=jnp.float32)
        i = jax.nn.sigmoid(g[:, 0:H])
        f = jax.nn.sigmoid(g[:, H:2 * H])
        gg = jnp.tanh(g[:, 2 * H:3 * H])
        o = jax.nn.sigmoid(g[:, 3 * H:4 * H])
        c = f * c + i * gg
        h = o * jnp.tanh(c)
    out_ref[...] = h


def _attn_body(po_ref, nopt_ref, wqT_ref, bq_ref, wkT_ref, wvT_ref, bk_ref,
               bv_ref, woT_ref, bo_ref, out_ref):
    po = po_ref[...]
    k = jnp.dot(po, wkT_ref[...], precision=_PREC) + bk_ref[...]
    v = jnp.dot(po, wvT_ref[...], precision=_PREC) + bv_ref[...]
    q = jnp.dot(nopt_ref[...], wqT_ref[...], precision=_PREC) + bq_ref[...]
    k3 = k.reshape(B, P, H)
    scores = jnp.sum(q[:, None, :] * k3, axis=-1) * 0.125
    m = jnp.max(scores, axis=-1, keepdims=True)
    e = jnp.exp(scores - m)
    w = e / jnp.sum(e, axis=-1, keepdims=True)
    attn = jnp.sum(w[:, :, None] * v.reshape(B, P, H), axis=1)
    out_ref[...] = jnp.dot(attn, woT_ref[...], precision=_PREC) + bo_ref[...]


def kernel(graph_embed, paths_nodes, paths_rels, node_opt, rel_embeds,
           W_ih, W_hh, b_ih, b_hh, in_proj_w, in_proj_b, out_proj_w, out_proj_b):
    graph_flat = graph_embed.reshape(B * N, D)
    idx_n = (jnp.transpose(paths_nodes, (2, 0, 1)).astype(jnp.int32)
             + (jnp.arange(B, dtype=jnp.int32) * N)[None, :, None])
    idx_n = idx_n.reshape(ITEMS // CH, CH)
    idx_r = jnp.transpose(paths_rels, (2, 0, 1)).astype(jnp.int32)
    idx_r = idx_r.reshape(ITEMS // CH, CH)

    x_seq = _make_sc_gather()(graph_flat, rel_embeds, idx_n, idx_r)
    x_seq = x_seq.reshape(L, BP, 2 * D)

    wT = W_ih.T.astype(jnp.bfloat16)
    whhT = W_hh.T.astype(jnp.bfloat16)
    bias = (b_ih + b_hh)[None, :]

    paths_out = pl.pallas_call(
        _lstm_body,
        grid=(BP // BLK,),
        in_specs=[
            pl.BlockSpec((L, BLK, 2 * D), lambda i: (0, i, 0)),
            pl.BlockSpec((2 * D, 4 * H), lambda i: (0, 0)),
            pl.BlockSpec((H, 4 * H), lambda i: (0, 0)),
            pl.BlockSpec((1, 4 * H), lambda i: (0, 0)),
        ],
        out_specs=pl.BlockSpec((BLK, H), lambda i: (i, 0)),
        out_shape=jax.ShapeDtypeStruct((BP, H), jnp.float32),
    )(x_seq, wT, whhT, bias)

    Wq, Wk, Wv = in_proj_w[:H], in_proj_w[H:2 * H], in_proj_w[2 * H:]
    bq, bk, bv = in_proj_b[:H], in_proj_b[H:2 * H], in_proj_b[2 * H:]

    out = pl.pallas_call(
        _attn_body,
        in_specs=[
            pl.BlockSpec((BP, H), lambda: (0, 0)),
            pl.BlockSpec((B, H), lambda: (0, 0)),
            pl.BlockSpec((H, H), lambda: (0, 0)),
            pl.BlockSpec((1, H), lambda: (0, 0)),
            pl.BlockSpec((H, H), lambda: (0, 0)),
            pl.BlockSpec((H, H), lambda: (0, 0)),
            pl.BlockSpec((1, H), lambda: (0, 0)),
            pl.BlockSpec((1, H), lambda: (0, 0)),
            pl.BlockSpec((H, H), lambda: (0, 0)),
            pl.BlockSpec((1, H), lambda: (0, 0)),
        ],
        out_specs=pl.BlockSpec((B, H), lambda: (0, 0)),
        out_shape=jax.ShapeDtypeStruct((B, H), jnp.float32),
    )(paths_out, node_opt, Wq.T, bq[None, :], Wk.T, Wv.T, bk[None, :],
      bv[None, :], out_proj_w.T, out_proj_b[None, :])
    return out
